# Initial kernel scaffold; baseline (speedup 1.0000x reference)
import jax
import jax.numpy as jnp
from jax.experimental import pallas as pl


def kernel(inputs, table, W, b):
    raise NotImplementedError("write your pallas kernel here")


# speedup vs baseline: 16.2873x; 16.2873x over previous
import functools

import jax
import jax.numpy as jnp
from jax import lax
from jax.experimental import pallas as pl
from jax.experimental.pallas import tpu as pltpu
from jax.experimental.pallas import tpu_sc as plsc

_LANE = 128
_FIRE = 13


@functools.lru_cache(maxsize=None)
def _build_gather(num_idx, d):
    info = plsc.get_sparse_core_info()
    nc, ns = info.num_cores, info.num_subcores
    nw = nc * ns
    per_w = num_idx // nw
    assert per_w * nw == num_idx
    n_stream = per_w // _LANE
    assert n_stream * _LANE == per_w
    n_super = n_stream // _FIRE
    assert n_super * _FIRE == n_stream
    chunk = _FIRE * _LANE

    mesh = plsc.VectorSubcoreMesh(core_axis_name="c", subcore_axis_name="s")

    @functools.partial(
        pl.kernel,
        mesh=mesh,
        compiler_params=pltpu.CompilerParams(use_tc_tiling_on_sc=False),
        out_type=jax.ShapeDtypeStruct((num_idx, d), jnp.float32),
        scratch_types=[
            pltpu.VMEM((n_stream, _LANE), jnp.int32),
            pltpu.VMEM((chunk, d), jnp.float32),
            pltpu.SemaphoreType.DMA,
        ],
    )
    def gather_k(idx_hbm, table_hbm, out_hbm, idx_v, rows_v, sem):
        wid = lax.axis_index("s") * nc + lax.axis_index("c")
        base = wid * per_w
        pltpu.sync_copy(idx_hbm.at[wid], idx_v)

        def superchunk(s, carry):
            for j in range(_FIRE):
                pltpu.make_async_copy(
                    table_hbm.at[idx_v.at[s * _FIRE + j]],
                    rows_v.at[pl.ds(j * _LANE, _LANE)],
                    sem,
                ).start()
            for j in range(_FIRE):
                pltpu.make_async_copy(
                    table_hbm.at[idx_v.at[s * _FIRE + j]],
                    rows_v.at[pl.ds(j * _LANE, _LANE)],
                    sem,
                ).wait()
            pltpu.sync_copy(rows_v, out_hbm.at[pl.ds(base + s * chunk, chunk)])
            return carry

        lax.fori_loop(0, n_super, superchunk, 0)

    return gather_k, nw, n_stream


@functools.lru_cache(maxsize=None)
def _build_matmul(bsz, k, d, blk):
    def mm_k(x_ref, w_ref, b_ref, o_ref):
        o_ref[...] = (
            jnp.dot(x_ref[...], w_ref[...], preferred_element_type=jnp.float32)
            + b_ref[...]
        )

    return pl.pallas_call(
        mm_k,
        grid=(bsz // blk,),
        in_specs=[
            pl.BlockSpec((blk, k), lambda i: (i, 0)),
            pl.BlockSpec((k, d), lambda i: (0, 0)),
            pl.BlockSpec((1, d), lambda i: (0, 0)),
        ],
        out_specs=pl.BlockSpec((blk, d), lambda i: (i, 0)),
        out_shape=jax.ShapeDtypeStruct((bsz, d), jnp.float32),
    )


def kernel(inputs, table, W, b):
    bsz, f = inputs.shape
    d = table.shape[1]
    num_idx = bsz * f
    gather_fn, nw, n_stream = _build_gather(num_idx, d)
    idx3 = inputs.astype(jnp.int32).reshape(nw, n_stream, _LANE)
    rows = gather_fn(idx3, table)
    x = rows.reshape(bsz, f * d)
    mm = _build_matmul(bsz, f * d, d, 512)
    return mm(x, W, b.reshape(1, d))
